# baseline (speedup 1.0000x reference)
import functools

import jax
import jax.numpy as jnp
from jax import lax
from jax.experimental import pallas as pl
from jax.experimental.pallas import tpu as pltpu
from jax.experimental.pallas import tpu_sc as plsc

N = 65536
E = 1048576
NC = 2
NS = 16
L = 16
N2 = N // 2

_MESH = dict(core_axis_name="c", subcore_axis_name="s")


def _zero_vmem_1d(buf, n):
    def body(i, _):
        buf[pl.ds(i * L, L)] = jnp.zeros((L,), jnp.float32)
        return 0
    lax.fori_loop(0, n // L, body, 0)


def _zero_vmem_2d(buf, rows, cols):
    def body(i, _):
        r = i // (cols // L)
        cc = i % (cols // L)
        buf[r, pl.ds(cc * L, L)] = jnp.zeros((L,), jnp.float32)
        return 0
    lax.fori_loop(0, rows * (cols // L), body, 0)


def _sc_prop_f1(v, ei, grow, srow):
    EPC = E // NC
    EPW = EPC // NS
    B = 128
    NCH = EPW // B
    ACC = N + 8
    ZT = N // NS

    @functools.partial(
        pl.kernel,
        out_type=jax.ShapeDtypeStruct((NC, N), jnp.float32),
        mesh=plsc.VectorSubcoreMesh(**_MESH),
        scratch_types=[
            pltpu.VMEM((N,), jnp.float32),
            pltpu.VMEM((ZT,), jnp.float32),
            pltpu.VMEM((B,), jnp.int32),
            pltpu.VMEM((B,), jnp.int32),
            pltpu.VMEM((B,), jnp.float32),
            pltpu.VMEM_SHARED((ACC,), jnp.float32),
        ],
        compiler_params=pltpu.CompilerParams(needs_layout_passes=False),
    )
    def k(v_hbm, ei_hbm, out_hbm, v_v, z_v, gi_v, si_v, vals_v, acc_sh):
        c = lax.axis_index("c")
        s = lax.axis_index("s")
        _zero_vmem_1d(z_v, ZT)
        pltpu.sync_copy(z_v, acc_sh.at[pl.ds(s * ZT, ZT)])
        pltpu.sync_copy(v_hbm, v_v)
        plsc.subcore_barrier()

        base = c * EPC + s * EPW

        def chunk(i, _):
            off = base + i * B
            pltpu.sync_copy(ei_hbm.at[grow, pl.ds(off, B)], gi_v)
            pltpu.sync_copy(ei_hbm.at[srow, pl.ds(off, B)], si_v)
            for j in range(B // L):
                g16 = gi_v[pl.ds(j * L, L)]
                s16 = si_v[pl.ds(j * L, L)]
                vals_v[pl.ds(j * L, L)] = plsc.load_gather(v_v, [g16])
                si_v[pl.ds(j * L, L)] = jnp.where(g16 == s16, N, s16)
            pltpu.sync_copy(vals_v, acc_sh.at[si_v], add=True)
            return 0

        lax.fori_loop(0, NCH, chunk, 0)
        plsc.subcore_barrier()
        pltpu.sync_copy(acc_sh.at[pl.ds(s * ZT, ZT)],
                        out_hbm.at[c, pl.ds(s * ZT, ZT)])

    return k(v, ei)


def _sc_prop_f64(u2n, ei):
    EPW = E // NS
    B = 128
    NB = 4
    SCH = NB * B
    NIT = EPW // (2 * SCH)
    F = 32
    ACCR = N2 + 8
    ZR = N2 // NS
    ZB = 512

    @functools.partial(
        pl.kernel,
        out_type=jax.ShapeDtypeStruct((NC, 2, N2, F), jnp.float32),
        mesh=plsc.VectorSubcoreMesh(**_MESH),
        scratch_types=[
            pltpu.VMEM((ZB, F), jnp.float32),
            pltpu.VMEM((SCH,), jnp.int32),
            pltpu.VMEM((SCH,), jnp.int32),
            pltpu.VMEM((SCH,), jnp.int32),
            pltpu.VMEM((SCH,), jnp.int32),
            pltpu.VMEM((NB, B), jnp.int32),
            pltpu.VMEM((NB, B), jnp.int32),
            pltpu.VMEM((NB, B), jnp.int32),
            pltpu.VMEM((NB, B), jnp.int32),
            pltpu.VMEM((SCH, F), jnp.float32),
            pltpu.VMEM((SCH, F), jnp.float32),
            pltpu.VMEM_SHARED((ACCR, F), jnp.float32),
            pltpu.SemaphoreType.DMA,
            pltpu.SemaphoreType.DMA,
            pltpu.SemaphoreType.DMA,
            pltpu.SemaphoreType.DMA,
            pltpu.SemaphoreType.DMA,
            pltpu.SemaphoreType.DMA,
        ],
        compiler_params=pltpu.CompilerParams(needs_layout_passes=False,
                                             use_tc_tiling_on_sc=False),
    )
    def k(u_hbm, ei_hbm, out_hbm, z_v, rsA, rdA, rsB, rdB,
          giA, siA, giB, siB, rwA, rwB, acc_sh,
          isA, isB, gsA, gsB, ssA, ssB):
        c = lax.axis_index("c")
        s = lax.axis_index("s")
        _zero_vmem_2d(z_v, ZB, F)
        for q in range(ZR // ZB):
            pltpu.sync_copy(z_v, acc_sh.at[pl.ds(s * ZR + q * ZB, ZB)])
        plsc.subcore_barrier()

        def stage(off, rs, rd, isem):
            pltpu.async_copy(ei_hbm.at[0, pl.ds(off, SCH)], rs, isem)
            pltpu.async_copy(ei_hbm.at[1, pl.ds(off, SCH)], rd, isem)

        def wait_stage(rs, rd, isem):
            pltpu.make_async_copy(ei_hbm.at[0, pl.ds(0, SCH)], rs, isem).wait()
            pltpu.make_async_copy(ei_hbm.at[1, pl.ds(0, SCH)], rd, isem).wait()

        for p in range(2):
            lo = p * N2

            def adjust(rs, rd, gi, si):
                for j in range(SCH // L):
                    g16 = rs[pl.ds(j * L, L)]
                    d16 = rd[pl.ds(j * L, L)]
                    ok = (d16 >= lo) & (d16 < lo + N2) & (g16 != d16)
                    r, cc = j // (B // L), (j % (B // L)) * L
                    gi[r, pl.ds(cc, L)] = g16 + c * N
                    si[r, pl.ds(cc, L)] = jnp.where(ok, d16 - lo, N2)

            def drain_scat(rw, si, ssem):
                for b in range(NB):
                    pltpu.make_async_copy(rw.at[pl.ds(b * B, B)],
                                          acc_sh.at[si.at[b]], ssem).wait()

            def fire_gathers(gi, rw, gsem):
                return [pltpu.async_copy(u_hbm.at[gi.at[b]],
                                         rw.at[pl.ds(b * B, B)], gsem)
                        for b in range(NB)]

            def fire_scats(rw, si, ssem):
                for b in range(NB):
                    pltpu.async_copy(rw.at[pl.ds(b * B, B)],
                                     acc_sh.at[si.at[b]], ssem, add=True)

            stage(s * EPW, rsA, rdA, isA)

            def pair(i, _):
                base = s * EPW + i * (2 * SCH)
                stage(base + SCH, rsB, rdB, isB)
                wait_stage(rsA, rdA, isA)
                @pl.when(i > 0)
                def _():
                    drain_scat(rwA, siA, ssA)
                adjust(rsA, rdA, giA, siA)
                gdA = fire_gathers(giA, rwA, gsA)
                @pl.when(i + 1 < NIT)
                def _():
                    stage(base + 2 * SCH, rsA, rdA, isA)
                wait_stage(rsB, rdB, isB)
                @pl.when(i > 0)
                def _():
                    drain_scat(rwB, siB, ssB)
                adjust(rsB, rdB, giB, siB)
                gdB = fire_gathers(giB, rwB, gsB)
                for d in gdA:
                    d.wait()
                fire_scats(rwA, siA, ssA)
                for d in gdB:
                    d.wait()
                fire_scats(rwB, siB, ssB)
                return 0

            lax.fori_loop(0, NIT, pair, 0)
            drain_scat(rwA, siA, ssA)
            drain_scat(rwB, siB, ssB)
            plsc.subcore_barrier()
            pltpu.sync_copy(acc_sh.at[pl.ds(s * ZR, ZR)],
                            out_hbm.at[c, p, pl.ds(s * ZR, ZR)])
            if p == 0:
                for q in range(ZR // ZB):
                    pltpu.sync_copy(z_v, acc_sh.at[pl.ds(s * ZR + q * ZB, ZB)])
                plsc.subcore_barrier()

    return k(u2n, ei)


def _tc_prep1(degp, x_flat):
    def body(degp_ref, x_ref, d_ref, d2_ref, u1_ref):
        deg = degp_ref[0] + degp_ref[1]
        d = jnp.where(deg > 0.0, lax.rsqrt(jnp.maximum(deg, 1e-12)), 0.0)
        d_ref[...] = d
        d2_ref[...] = d * d
        u1_ref[...] = d * x_ref[...]

    sh = jax.ShapeDtypeStruct((512, 128), jnp.float32)
    return pl.pallas_call(body, out_shape=(sh, sh, sh))(degp, x_flat)


def _tc_prep2(a1p, d, d2):
    def body(a1p_ref, d_ref, d2_ref, u2_ref, da1_ref):
        a1 = a1p_ref[0] + a1p_ref[1]
        u2_ref[...] = d2_ref[...] * a1
        da1_ref[...] = d_ref[...] * a1

    sh = jax.ShapeDtypeStruct((512, 128), jnp.float32)
    return pl.pallas_call(body, out_shape=(sh, sh))(a1p, d, d2)


def _tc_layer1(x, da1, a2p, d, W1, b1):
    BN = 4096
    G = N // BN

    def body(x_ref, da1_ref, a2p_ref, d_ref, W1_ref, b1_ref, h1_ref, u_ref):
        w0 = W1_ref[0]
        w1 = W1_ref[1]
        w2 = W1_ref[2]
        d = d_ref[...]
        xb = x_ref[...]
        tx1 = -da1_ref[...]
        tx2 = 2.0 * (d * (a2p_ref[0] + a2p_ref[1])) - xb
        out = xb * w0 + tx1 * w1 + tx2 * w2 + b1_ref[...]
        h1 = jnp.maximum(out, 0.0)
        h1_ref[...] = h1
        dh = d * h1
        u_ref[0] = dh[:, :32]
        u_ref[1] = dh[:, 32:]

    return pl.pallas_call(
        body,
        grid=(G,),
        in_specs=[
            pl.BlockSpec((BN, 1), lambda i: (i, 0)),
            pl.BlockSpec((BN, 1), lambda i: (i, 0)),
            pl.BlockSpec((2, BN, 1), lambda i: (0, i, 0)),
            pl.BlockSpec((BN, 1), lambda i: (i, 0)),
            pl.BlockSpec((3, 1, 64), lambda i: (0, 0, 0)),
            pl.BlockSpec((1, 64), lambda i: (0, 0)),
        ],
        out_specs=[
            pl.BlockSpec((BN, 64), lambda i: (i, 0)),
            pl.BlockSpec((2, BN, 32), lambda i: (0, i, 0)),
        ],
        out_shape=[
            jax.ShapeDtypeStruct((N, 64), jnp.float32),
            jax.ShapeDtypeStruct((2, N, 32), jnp.float32),
        ],
    )(x, da1, a2p, d, W1, b1)


def _tc_prep3(c1p, d, d2):
    BN = 4096
    G = N // BN
    BH = N2 // BN

    def body(c1p_ref, d_ref, d2_ref, u_ref, dc1_ref):
        lo = c1p_ref[0, 0]
        hi = c1p_ref[1, 0]
        c1 = jnp.concatenate([lo, hi], axis=1)
        dc1_ref[...] = d_ref[...] * c1
        u2 = d2_ref[...] * c1
        u_ref[0] = u2[:, :32]
        u_ref[1] = u2[:, 32:]

    return pl.pallas_call(
        body,
        grid=(G,),
        in_specs=[
            pl.BlockSpec((2, 1, BN, 32), lambda i: (0, i // BH, i % BH, 0)),
            pl.BlockSpec((BN, 1), lambda i: (i, 0)),
            pl.BlockSpec((BN, 1), lambda i: (i, 0)),
        ],
        out_specs=[
            pl.BlockSpec((2, BN, 32), lambda i: (0, i, 0)),
            pl.BlockSpec((BN, 64), lambda i: (i, 0)),
        ],
        out_shape=[
            jax.ShapeDtypeStruct((2, N, 32), jnp.float32),
            jax.ShapeDtypeStruct((N, 64), jnp.float32),
        ],
    )(c1p, d, d2)


def _tc_final(h1, dc1, c2p, d, W2, b2):
    BN = 4096
    G = N // BN
    BH = N2 // BN

    def body(h1_ref, dc1_ref, c2p_ref, d_ref, W2_ref, b2_ref, m_ref):
        lo = c2p_ref[0, 0]
        hi = c2p_ref[1, 0]
        h1b = h1_ref[...]
        tx1 = -dc1_ref[...]
        tx2 = 2.0 * (d_ref[...] * jnp.concatenate([lo, hi], axis=1)) - h1b
        w0 = W2_ref[0]
        w1 = W2_ref[1]
        w2 = W2_ref[2]
        out = jnp.dot(h1b, w0, preferred_element_type=jnp.float32)
        out = out + jnp.dot(tx1, w1, preferred_element_type=jnp.float32)
        out = out + jnp.dot(tx2, w2, preferred_element_type=jnp.float32)
        out = out + b2_ref[...]
        h2 = jnp.maximum(out, 0.0)
        m_ref[...] = jnp.max(h2, axis=1, keepdims=True)

    return pl.pallas_call(
        body,
        grid=(G,),
        in_specs=[
            pl.BlockSpec((BN, 64), lambda i: (i, 0)),
            pl.BlockSpec((BN, 64), lambda i: (i, 0)),
            pl.BlockSpec((2, 1, BN, 32), lambda i: (0, i // BH, i % BH, 0)),
            pl.BlockSpec((BN, 1), lambda i: (i, 0)),
            pl.BlockSpec((3, 64, 128), lambda i: (0, 0, 0)),
            pl.BlockSpec((1, 128), lambda i: (0, 0)),
        ],
        out_specs=pl.BlockSpec((BN, 1), lambda i: (i, 0)),
        out_shape=jax.ShapeDtypeStruct((N, 1), jnp.float32),
    )(h1, dc1, c2p, d, W2, b2)


def _tc_head(m2, fc1_W, fc1_b, fc2_W, fc2_b):
    def body(m_ref, w1_ref, b1_ref, w2_ref, b2_ref, o_ref):
        h = jnp.dot(m_ref[...], w1_ref[...], preferred_element_type=jnp.float32)
        h = jnp.maximum(h + b1_ref[...], 0.0)
        o_ref[...] = jnp.dot(h, w2_ref[...],
                             preferred_element_type=jnp.float32) + b2_ref[...]

    return pl.pallas_call(
        body, out_shape=jax.ShapeDtypeStruct((512, 1), jnp.float32),
    )(m2, fc1_W, fc1_b, fc2_W, fc2_b)


def kernel(x, edge_index, W1, b1, W2, b2, fc1_W, fc1_b, fc2_W, fc2_b):
    ei = edge_index
    x1 = x.reshape(512, 128)

    degp = _sc_prop_f1(jnp.ones((N,), jnp.float32), ei, grow=1, srow=0)
    d_f, d2_f, u1_f = _tc_prep1(degp.reshape(2, 512, 128), x1)

    a1p = _sc_prop_f1(u1_f.reshape(N), ei, grow=0, srow=1)
    u2_f, da1_f = _tc_prep2(a1p.reshape(2, 512, 128), d_f, d2_f)

    a2p = _sc_prop_f1(u2_f.reshape(N), ei, grow=0, srow=1)

    d_c = d_f.reshape(N, 1)
    h1, U1 = _tc_layer1(x, da1_f.reshape(N, 1), a2p.reshape(2, N, 1),
                        d_c, W1, b1.reshape(1, 64))

    c1p = _sc_prop_f64(U1.reshape(2 * N, 32), ei)
    U2, dc1 = _tc_prep3(c1p, d_c, d2_f.reshape(N, 1))

    c2p = _sc_prop_f64(U2.reshape(2 * N, 32), ei)

    m = _tc_final(h1, dc1, c2p, d_c, W2, b2.reshape(1, 128))
    return _tc_head(m.reshape(512, 128), fc1_W, fc1_b.reshape(1, 64),
                    fc2_W, fc2_b.reshape(1, 1))


# speedup vs baseline: 1.6794x; 1.6794x over previous
import functools

import jax
import jax.numpy as jnp
from jax import lax
from jax.experimental import pallas as pl
from jax.experimental.pallas import tpu as pltpu
from jax.experimental.pallas import tpu_sc as plsc

N = 65536
E = 1048576
NC = 2
NS = 16
L = 16
N2 = N // 2

_MESH = dict(core_axis_name="c", subcore_axis_name="s")


def _zero_vmem_1d(buf, n):
    def body(i, _):
        buf[pl.ds(i * L, L)] = jnp.zeros((L,), jnp.float32)
        return 0
    lax.fori_loop(0, n // L, body, 0)


def _zero_vmem_2d(buf, rows, cols):
    def body(i, _):
        r = i // (cols // L)
        cc = i % (cols // L)
        buf[r, pl.ds(cc * L, L)] = jnp.zeros((L,), jnp.float32)
        return 0
    lax.fori_loop(0, rows * (cols // L), body, 0)


def _sc_prop_f1(v, ei, grow, srow):
    EPC = E // NC
    EPW = EPC // NS
    B = 128
    NCH = EPW // B
    ACC = N + 8
    ZT = N // NS

    @functools.partial(
        pl.kernel,
        out_type=jax.ShapeDtypeStruct((NC, N), jnp.float32),
        mesh=plsc.VectorSubcoreMesh(**_MESH),
        scratch_types=[
            pltpu.VMEM((N,), jnp.float32),
            pltpu.VMEM((ZT,), jnp.float32),
            pltpu.VMEM((B,), jnp.int32),
            pltpu.VMEM((B,), jnp.int32),
            pltpu.VMEM((B,), jnp.float32),
            pltpu.VMEM_SHARED((ACC,), jnp.float32),
        ],
        compiler_params=pltpu.CompilerParams(needs_layout_passes=False),
    )
    def k(v_hbm, ei_hbm, out_hbm, v_v, z_v, gi_v, si_v, vals_v, acc_sh):
        c = lax.axis_index("c")
        s = lax.axis_index("s")
        _zero_vmem_1d(z_v, ZT)
        pltpu.sync_copy(z_v, acc_sh.at[pl.ds(s * ZT, ZT)])
        pltpu.sync_copy(v_hbm, v_v)
        plsc.subcore_barrier()

        base = c * EPC + s * EPW

        def chunk(i, _):
            off = base + i * B
            pltpu.sync_copy(ei_hbm.at[grow, pl.ds(off, B)], gi_v)
            pltpu.sync_copy(ei_hbm.at[srow, pl.ds(off, B)], si_v)
            for j in range(B // L):
                g16 = gi_v[pl.ds(j * L, L)]
                s16 = si_v[pl.ds(j * L, L)]
                vals_v[pl.ds(j * L, L)] = plsc.load_gather(v_v, [g16])
                si_v[pl.ds(j * L, L)] = jnp.where(g16 == s16, N, s16)
            pltpu.sync_copy(vals_v, acc_sh.at[si_v], add=True)
            return 0

        lax.fori_loop(0, NCH, chunk, 0)
        plsc.subcore_barrier()
        pltpu.sync_copy(acc_sh.at[pl.ds(s * ZT, ZT)],
                        out_hbm.at[c, pl.ds(s * ZT, ZT)])

    return k(v, ei)


def _sc_prop_f64(u4n, ei):
    EPW = E // NS
    B = 128
    NB = 4
    SCH = NB * B
    NIT = EPW // (2 * SCH)
    F = 16
    ZR = N // NS
    ZB = 512

    @functools.partial(
        pl.kernel,
        out_type=jax.ShapeDtypeStruct((NC, 2, N, F), jnp.float32),
        mesh=plsc.VectorSubcoreMesh(**_MESH),
        scratch_types=[
            pltpu.VMEM((ZB, F), jnp.float32),
            pltpu.VMEM((SCH,), jnp.int32),
            pltpu.VMEM((SCH,), jnp.int32),
            pltpu.VMEM((SCH,), jnp.int32),
            pltpu.VMEM((SCH,), jnp.int32),
            pltpu.VMEM((NB, B), jnp.int32),
            pltpu.VMEM((NB, B), jnp.int32),
            pltpu.VMEM((NB, B), jnp.int32),
            pltpu.VMEM((NB, B), jnp.int32),
            pltpu.VMEM((SCH, F), jnp.float32),
            pltpu.VMEM((SCH, F), jnp.float32),
            pltpu.VMEM_SHARED((N, F), jnp.float32),
            pltpu.SemaphoreType.DMA,
            pltpu.SemaphoreType.DMA,
            pltpu.SemaphoreType.DMA,
            pltpu.SemaphoreType.DMA,
            pltpu.SemaphoreType.DMA,
            pltpu.SemaphoreType.DMA,
        ],
        compiler_params=pltpu.CompilerParams(needs_layout_passes=False,
                                             use_tc_tiling_on_sc=False),
    )
    def k(u_hbm, ei_hbm, out_hbm, z_v, rsA, rdA, rsB, rdB,
          giA, siA, giB, siB, rwA, rwB, acc_sh,
          isA, isB, gsA, gsB, ssA, ssB):
        c = lax.axis_index("c")
        s = lax.axis_index("s")
        _zero_vmem_2d(z_v, ZB, F)
        for q in range(ZR // ZB):
            pltpu.sync_copy(z_v, acc_sh.at[pl.ds(s * ZR + q * ZB, ZB)])
        plsc.subcore_barrier()

        def stage(off, rs, rd, isem):
            pltpu.async_copy(ei_hbm.at[0, pl.ds(off, SCH)], rs, isem)
            pltpu.async_copy(ei_hbm.at[1, pl.ds(off, SCH)], rd, isem)

        def wait_stage(rs, rd, isem):
            pltpu.make_async_copy(ei_hbm.at[0, pl.ds(0, SCH)], rs, isem).wait()
            pltpu.make_async_copy(ei_hbm.at[1, pl.ds(0, SCH)], rd, isem).wait()

        for j in range(2):
            qoff = (2 * c + j) * N

            def adjust(rs, rd, gi, si):
                for t in range(SCH // L):
                    g16 = rs[pl.ds(t * L, L)]
                    d16 = rd[pl.ds(t * L, L)]
                    r, cc = t // (B // L), (t % (B // L)) * L
                    gi[r, pl.ds(cc, L)] = jnp.where(g16 == d16, 4 * N,
                                                    g16 + qoff)
                    si[r, pl.ds(cc, L)] = d16

            def drain_scat(rw, si, ssem):
                for b in range(NB):
                    pltpu.make_async_copy(rw.at[pl.ds(b * B, B)],
                                          acc_sh.at[si.at[b]], ssem).wait()

            def fire_gathers(gi, rw, gsem):
                return [pltpu.async_copy(u_hbm.at[gi.at[b]],
                                         rw.at[pl.ds(b * B, B)], gsem)
                        for b in range(NB)]

            def fire_scats(rw, si, ssem):
                for b in range(NB):
                    pltpu.async_copy(rw.at[pl.ds(b * B, B)],
                                     acc_sh.at[si.at[b]], ssem, add=True)

            stage(s * EPW, rsA, rdA, isA)

            def pair(i, _):
                base = s * EPW + i * (2 * SCH)
                stage(base + SCH, rsB, rdB, isB)
                wait_stage(rsA, rdA, isA)
                @pl.when(i > 0)
                def _():
                    drain_scat(rwA, siA, ssA)
                adjust(rsA, rdA, giA, siA)
                gdA = fire_gathers(giA, rwA, gsA)
                @pl.when(i + 1 < NIT)
                def _():
                    stage(base + 2 * SCH, rsA, rdA, isA)
                wait_stage(rsB, rdB, isB)
                @pl.when(i > 0)
                def _():
                    drain_scat(rwB, siB, ssB)
                adjust(rsB, rdB, giB, siB)
                gdB = fire_gathers(giB, rwB, gsB)
                for d in gdA:
                    d.wait()
                fire_scats(rwA, siA, ssA)
                for d in gdB:
                    d.wait()
                fire_scats(rwB, siB, ssB)
                return 0

            lax.fori_loop(0, NIT, pair, 0)
            drain_scat(rwA, siA, ssA)
            drain_scat(rwB, siB, ssB)
            plsc.subcore_barrier()
            pltpu.sync_copy(acc_sh.at[pl.ds(s * ZR, ZR)],
                            out_hbm.at[c, j, pl.ds(s * ZR, ZR)])
            if j == 0:
                for q in range(ZR // ZB):
                    pltpu.sync_copy(z_v, acc_sh.at[pl.ds(s * ZR + q * ZB, ZB)])
                plsc.subcore_barrier()

    return k(u4n, ei)


def _tc_prep1(degp, x_flat):
    def body(degp_ref, x_ref, d_ref, d2_ref, u1_ref):
        deg = degp_ref[0] + degp_ref[1]
        d = jnp.where(deg > 0.0, lax.rsqrt(jnp.maximum(deg, 1e-12)), 0.0)
        d_ref[...] = d
        d2_ref[...] = d * d
        u1_ref[...] = d * x_ref[...]

    sh = jax.ShapeDtypeStruct((512, 128), jnp.float32)
    return pl.pallas_call(body, out_shape=(sh, sh, sh))(degp, x_flat)


def _tc_prep2(a1p, d, d2):
    def body(a1p_ref, d_ref, d2_ref, u2_ref, da1_ref):
        a1 = a1p_ref[0] + a1p_ref[1]
        u2_ref[...] = d2_ref[...] * a1
        da1_ref[...] = d_ref[...] * a1

    sh = jax.ShapeDtypeStruct((512, 128), jnp.float32)
    return pl.pallas_call(body, out_shape=(sh, sh))(a1p, d, d2)


def _tc_layer1(x, da1, a2p, d, W1, b1):
    BN = 4096
    G = N // BN

    def body(x_ref, da1_ref, a2p_ref, d_ref, W1_ref, b1_ref, h1_ref, u_ref):
        w0 = W1_ref[0]
        w1 = W1_ref[1]
        w2 = W1_ref[2]
        d = d_ref[...]
        xb = x_ref[...]
        tx1 = -da1_ref[...]
        tx2 = 2.0 * (d * (a2p_ref[0] + a2p_ref[1])) - xb
        out = xb * w0 + tx1 * w1 + tx2 * w2 + b1_ref[...]
        h1 = jnp.maximum(out, 0.0)
        h1_ref[...] = h1
        dh = d * h1
        for q in range(4):
            u_ref[q] = dh[:, 16 * q:16 * (q + 1)]

    return pl.pallas_call(
        body,
        grid=(G,),
        in_specs=[
            pl.BlockSpec((BN, 1), lambda i: (i, 0)),
            pl.BlockSpec((BN, 1), lambda i: (i, 0)),
            pl.BlockSpec((2, BN, 1), lambda i: (0, i, 0)),
            pl.BlockSpec((BN, 1), lambda i: (i, 0)),
            pl.BlockSpec((3, 1, 64), lambda i: (0, 0, 0)),
            pl.BlockSpec((1, 64), lambda i: (0, 0)),
        ],
        out_specs=[
            pl.BlockSpec((BN, 64), lambda i: (i, 0)),
            pl.BlockSpec((4, BN, 16), lambda i: (0, i, 0)),
        ],
        out_shape=[
            jax.ShapeDtypeStruct((N, 64), jnp.float32),
            jax.ShapeDtypeStruct((4, N, 16), jnp.float32),
        ],
    )(x, da1, a2p, d, W1, b1)


def _asm16(cp_ref):
    return jnp.concatenate([cp_ref[0, 0], cp_ref[0, 1],
                            cp_ref[1, 0], cp_ref[1, 1]], axis=1)


def _tc_prep3(c1p, d, d2):
    BN = 4096
    G = N // BN

    def body(c1p_ref, d_ref, d2_ref, u_ref, dc1_ref):
        c1 = _asm16(c1p_ref)
        dc1_ref[...] = d_ref[...] * c1
        u2 = d2_ref[...] * c1
        for q in range(4):
            u_ref[q] = u2[:, 16 * q:16 * (q + 1)]

    return pl.pallas_call(
        body,
        grid=(G,),
        in_specs=[
            pl.BlockSpec((2, 2, BN, 16), lambda i: (0, 0, i, 0)),
            pl.BlockSpec((BN, 1), lambda i: (i, 0)),
            pl.BlockSpec((BN, 1), lambda i: (i, 0)),
        ],
        out_specs=[
            pl.BlockSpec((4, BN, 16), lambda i: (0, i, 0)),
            pl.BlockSpec((BN, 64), lambda i: (i, 0)),
        ],
        out_shape=[
            jax.ShapeDtypeStruct((4, N, 16), jnp.float32),
            jax.ShapeDtypeStruct((N, 64), jnp.float32),
        ],
    )(c1p, d, d2)


def _tc_final(h1, dc1, c2p, d, W2, b2):
    BN = 4096
    G = N // BN

    def body(h1_ref, dc1_ref, c2p_ref, d_ref, W2_ref, b2_ref, m_ref):
        h1b = h1_ref[...]
        tx1 = -dc1_ref[...]
        tx2 = 2.0 * (d_ref[...] * _asm16(c2p_ref)) - h1b
        w0 = W2_ref[0]
        w1 = W2_ref[1]
        w2 = W2_ref[2]
        out = jnp.dot(h1b, w0, preferred_element_type=jnp.float32)
        out = out + jnp.dot(tx1, w1, preferred_element_type=jnp.float32)
        out = out + jnp.dot(tx2, w2, preferred_element_type=jnp.float32)
        out = out + b2_ref[...]
        h2 = jnp.maximum(out, 0.0)
        m_ref[...] = jnp.max(h2, axis=1, keepdims=True)

    return pl.pallas_call(
        body,
        grid=(G,),
        in_specs=[
            pl.BlockSpec((BN, 64), lambda i: (i, 0)),
            pl.BlockSpec((BN, 64), lambda i: (i, 0)),
            pl.BlockSpec((2, 2, BN, 16), lambda i: (0, 0, i, 0)),
            pl.BlockSpec((BN, 1), lambda i: (i, 0)),
            pl.BlockSpec((3, 64, 128), lambda i: (0, 0, 0)),
            pl.BlockSpec((1, 128), lambda i: (0, 0)),
        ],
        out_specs=pl.BlockSpec((BN, 1), lambda i: (i, 0)),
        out_shape=jax.ShapeDtypeStruct((N, 1), jnp.float32),
    )(h1, dc1, c2p, d, W2, b2)


def _tc_head(m2, fc1_W, fc1_b, fc2_W, fc2_b):
    def body(m_ref, w1_ref, b1_ref, w2_ref, b2_ref, o_ref):
        h = jnp.dot(m_ref[...], w1_ref[...], preferred_element_type=jnp.float32)
        h = jnp.maximum(h + b1_ref[...], 0.0)
        o_ref[...] = jnp.dot(h, w2_ref[...],
                             preferred_element_type=jnp.float32) + b2_ref[...]

    return pl.pallas_call(
        body, out_shape=jax.ShapeDtypeStruct((512, 1), jnp.float32),
    )(m2, fc1_W, fc1_b, fc2_W, fc2_b)


def kernel(x, edge_index, W1, b1, W2, b2, fc1_W, fc1_b, fc2_W, fc2_b):
    ei = edge_index
    x1 = x.reshape(512, 128)

    degp = _sc_prop_f1(jnp.ones((N,), jnp.float32), ei, grow=1, srow=0)
    d_f, d2_f, u1_f = _tc_prep1(degp.reshape(2, 512, 128), x1)

    a1p = _sc_prop_f1(u1_f.reshape(N), ei, grow=0, srow=1)
    u2_f, da1_f = _tc_prep2(a1p.reshape(2, 512, 128), d_f, d2_f)

    a2p = _sc_prop_f1(u2_f.reshape(N), ei, grow=0, srow=1)

    d_c = d_f.reshape(N, 1)
    h1, U1 = _tc_layer1(x, da1_f.reshape(N, 1), a2p.reshape(2, N, 1),
                        d_c, W1, b1.reshape(1, 64))

    z8 = jnp.zeros((8, 16), jnp.float32)
    c1p = _sc_prop_f64(jnp.concatenate([U1.reshape(4 * N, 16), z8]), ei)
    U2, dc1 = _tc_prep3(c1p, d_c, d2_f.reshape(N, 1))

    c2p = _sc_prop_f64(jnp.concatenate([U2.reshape(4 * N, 16), z8]), ei)

    m = _tc_final(h1, dc1, c2p, d_c, W2, b2.reshape(1, 128))
    return _tc_head(m.reshape(512, 128), fc1_W, fc1_b.reshape(1, 64),
                    fc2_W, fc2_b.reshape(1, 1))


# speedup vs baseline: 2.4758x; 1.4742x over previous
import functools

import jax
import jax.numpy as jnp
from jax import lax
from jax.experimental import pallas as pl
from jax.experimental.pallas import tpu as pltpu
from jax.experimental.pallas import tpu_sc as plsc

N = 65536
E = 1048576
NC = 2
NS = 16
L = 16
N2 = N // 2

_MESH = dict(core_axis_name="c", subcore_axis_name="s")


def _zero_vmem_1d(buf, n):
    def body(i, _):
        buf[pl.ds(i * L, L)] = jnp.zeros((L,), jnp.float32)
        return 0
    lax.fori_loop(0, n // L, body, 0)


def _zero_vmem_2d(buf, rows, cols):
    def body(i, _):
        r = i // (cols // L)
        cc = i % (cols // L)
        buf[r, pl.ds(cc * L, L)] = jnp.zeros((L,), jnp.float32)
        return 0
    lax.fori_loop(0, rows * (cols // L), body, 0)


def _sc_prop_f1(v, ei, grow, srow):
    EPC = E // NC
    EPW = EPC // NS
    B = 128
    NCH = EPW // B
    ACC = N + 8
    ZT = N // NS

    NB = 4
    SCH = NB * B
    NIT = EPW // (2 * SCH)

    @functools.partial(
        pl.kernel,
        out_type=jax.ShapeDtypeStruct((NC, N), jnp.float32),
        mesh=plsc.VectorSubcoreMesh(**_MESH),
        scratch_types=[
            pltpu.VMEM((N,), jnp.float32),
            pltpu.VMEM((ZT,), jnp.float32),
            pltpu.VMEM((SCH,), jnp.int32),
            pltpu.VMEM((SCH,), jnp.int32),
            pltpu.VMEM((SCH,), jnp.int32),
            pltpu.VMEM((SCH,), jnp.int32),
            pltpu.VMEM((NB, B), jnp.int32),
            pltpu.VMEM((NB, B), jnp.int32),
            pltpu.VMEM((SCH,), jnp.float32),
            pltpu.VMEM((SCH,), jnp.float32),
            pltpu.VMEM_SHARED((ACC,), jnp.float32),
            pltpu.SemaphoreType.DMA,
            pltpu.SemaphoreType.DMA,
            pltpu.SemaphoreType.DMA,
            pltpu.SemaphoreType.DMA,
        ],
        compiler_params=pltpu.CompilerParams(needs_layout_passes=False),
    )
    def k(v_hbm, ei_hbm, out_hbm, v_v, z_v, rgA, rgB, rsA, rsB,
          siA, siB, vlA, vlB, acc_sh, isA, isB, ssA, ssB):
        c = lax.axis_index("c")
        s = lax.axis_index("s")
        _zero_vmem_1d(z_v, ZT)
        pltpu.sync_copy(z_v, acc_sh.at[pl.ds(s * ZT, ZT)])
        pltpu.sync_copy(v_hbm, v_v)
        plsc.subcore_barrier()

        base = c * EPC + s * EPW

        def stage(off, rg, rs, isem):
            pltpu.async_copy(ei_hbm.at[grow, pl.ds(off, SCH)], rg, isem)
            pltpu.async_copy(ei_hbm.at[srow, pl.ds(off, SCH)], rs, isem)

        def wait_stage(rg, rs, isem):
            pltpu.make_async_copy(ei_hbm.at[0, pl.ds(0, SCH)], rg, isem).wait()
            pltpu.make_async_copy(ei_hbm.at[0, pl.ds(0, SCH)], rs, isem).wait()

        def drain_scat(vl, si, ssem):
            for b in range(NB):
                pltpu.make_async_copy(vl.at[pl.ds(b * B, B)],
                                      acc_sh.at[si.at[b]], ssem).wait()

        def work(rg, rs, si, vl):
            for j in range(SCH // L):
                g16 = rg[pl.ds(j * L, L)]
                s16 = rs[pl.ds(j * L, L)]
                vl[pl.ds(j * L, L)] = plsc.load_gather(v_v, [g16])
                r, cc = j // (B // L), (j % (B // L)) * L
                si[r, pl.ds(cc, L)] = jnp.where(g16 == s16, N, s16)

        def fire_scats(vl, si, ssem):
            for b in range(NB):
                pltpu.async_copy(vl.at[pl.ds(b * B, B)],
                                 acc_sh.at[si.at[b]], ssem, add=True)

        stage(base, rgA, rsA, isA)

        def pair(i, _):
            off = base + i * (2 * SCH)
            stage(off + SCH, rgB, rsB, isB)
            wait_stage(rgA, rsA, isA)
            @pl.when(i > 0)
            def _():
                drain_scat(vlA, siA, ssA)
            work(rgA, rsA, siA, vlA)
            fire_scats(vlA, siA, ssA)
            @pl.when(i + 1 < NIT)
            def _():
                stage(off + 2 * SCH, rgA, rsA, isA)
            wait_stage(rgB, rsB, isB)
            @pl.when(i > 0)
            def _():
                drain_scat(vlB, siB, ssB)
            work(rgB, rsB, siB, vlB)
            fire_scats(vlB, siB, ssB)
            return 0

        lax.fori_loop(0, NIT, pair, 0)
        drain_scat(vlA, siA, ssA)
        drain_scat(vlB, siB, ssB)
        plsc.subcore_barrier()
        pltpu.sync_copy(acc_sh.at[pl.ds(s * ZT, ZT)],
                        out_hbm.at[c, pl.ds(s * ZT, ZT)])

    return k(v, ei)


def _sc_prop_f64(u4n, ei):
    EPW = E // NS
    B = 128
    NB = 4
    SCH = NB * B
    NIT = EPW // (2 * SCH)
    F = 16
    ZR = N // NS
    ZB = 512

    @functools.partial(
        pl.kernel,
        out_type=jax.ShapeDtypeStruct((NC, 2, N, F), jnp.float32),
        mesh=plsc.VectorSubcoreMesh(**_MESH),
        scratch_types=[
            pltpu.VMEM((ZB, F), jnp.float32),
            pltpu.VMEM((SCH,), jnp.int32),
            pltpu.VMEM((SCH,), jnp.int32),
            pltpu.VMEM((SCH,), jnp.int32),
            pltpu.VMEM((SCH,), jnp.int32),
            pltpu.VMEM((NB, B), jnp.int32),
            pltpu.VMEM((NB, B), jnp.int32),
            pltpu.VMEM((NB, B), jnp.int32),
            pltpu.VMEM((NB, B), jnp.int32),
            pltpu.VMEM((SCH, F), jnp.float32),
            pltpu.VMEM((SCH, F), jnp.float32),
            pltpu.VMEM_SHARED((N, F), jnp.float32),
            pltpu.SemaphoreType.DMA,
            pltpu.SemaphoreType.DMA,
            pltpu.SemaphoreType.DMA,
            pltpu.SemaphoreType.DMA,
            pltpu.SemaphoreType.DMA,
            pltpu.SemaphoreType.DMA,
        ],
        compiler_params=pltpu.CompilerParams(needs_layout_passes=False,
                                             use_tc_tiling_on_sc=False),
    )
    def k(u_hbm, ei_hbm, out_hbm, z_v, rsA, rdA, rsB, rdB,
          giA, siA, giB, siB, rwA, rwB, acc_sh,
          isA, isB, gsA, gsB, ssA, ssB):
        c = lax.axis_index("c")
        s = lax.axis_index("s")
        _zero_vmem_2d(z_v, ZB, F)
        for q in range(ZR // ZB):
            pltpu.sync_copy(z_v, acc_sh.at[pl.ds(s * ZR + q * ZB, ZB)])
        plsc.subcore_barrier()

        def stage(off, rs, rd, isem):
            pltpu.async_copy(ei_hbm.at[0, pl.ds(off, SCH)], rs, isem)
            pltpu.async_copy(ei_hbm.at[1, pl.ds(off, SCH)], rd, isem)

        def wait_stage(rs, rd, isem):
            pltpu.make_async_copy(ei_hbm.at[0, pl.ds(0, SCH)], rs, isem).wait()
            pltpu.make_async_copy(ei_hbm.at[1, pl.ds(0, SCH)], rd, isem).wait()

        for j in range(2):
            qoff = (2 * c + j) * N

            def adjust(rs, rd, gi, si):
                for t in range(SCH // L):
                    g16 = rs[pl.ds(t * L, L)]
                    d16 = rd[pl.ds(t * L, L)]
                    r, cc = t // (B // L), (t % (B // L)) * L
                    gi[r, pl.ds(cc, L)] = jnp.where(g16 == d16, 4 * N,
                                                    g16 + qoff)
                    si[r, pl.ds(cc, L)] = d16

            def drain_scat(rw, si, ssem):
                for b in range(NB):
                    pltpu.make_async_copy(rw.at[pl.ds(b * B, B)],
                                          acc_sh.at[si.at[b]], ssem).wait()

            def fire_gathers(gi, rw, gsem):
                return [pltpu.async_copy(u_hbm.at[gi.at[b]],
                                         rw.at[pl.ds(b * B, B)], gsem)
                        for b in range(NB)]

            def fire_scats(rw, si, ssem):
                for b in range(NB):
                    pltpu.async_copy(rw.at[pl.ds(b * B, B)],
                                     acc_sh.at[si.at[b]], ssem, add=True)

            stage(s * EPW, rsA, rdA, isA)

            def pair(i, _):
                base = s * EPW + i * (2 * SCH)
                stage(base + SCH, rsB, rdB, isB)
                wait_stage(rsA, rdA, isA)
                @pl.when(i > 0)
                def _():
                    drain_scat(rwA, siA, ssA)
                adjust(rsA, rdA, giA, siA)
                gdA = fire_gathers(giA, rwA, gsA)
                @pl.when(i + 1 < NIT)
                def _():
                    stage(base + 2 * SCH, rsA, rdA, isA)
                wait_stage(rsB, rdB, isB)
                @pl.when(i > 0)
                def _():
                    drain_scat(rwB, siB, ssB)
                adjust(rsB, rdB, giB, siB)
                gdB = fire_gathers(giB, rwB, gsB)
                for d in gdA:
                    d.wait()
                fire_scats(rwA, siA, ssA)
                for d in gdB:
                    d.wait()
                fire_scats(rwB, siB, ssB)
                return 0

            lax.fori_loop(0, NIT, pair, 0)
            drain_scat(rwA, siA, ssA)
            drain_scat(rwB, siB, ssB)
            plsc.subcore_barrier()
            pltpu.sync_copy(acc_sh.at[pl.ds(s * ZR, ZR)],
                            out_hbm.at[c, j, pl.ds(s * ZR, ZR)])
            if j == 0:
                for q in range(ZR // ZB):
                    pltpu.sync_copy(z_v, acc_sh.at[pl.ds(s * ZR + q * ZB, ZB)])
                plsc.subcore_barrier()

    return k(u4n, ei)


def _tc_prep1(degp, x_flat):
    def body(degp_ref, x_ref, d_ref, d2_ref, u1_ref):
        deg = degp_ref[0] + degp_ref[1]
        d = jnp.where(deg > 0.0, lax.rsqrt(jnp.maximum(deg, 1e-12)), 0.0)
        d_ref[...] = d
        d2_ref[...] = d * d
        u1_ref[...] = d * x_ref[...]

    sh = jax.ShapeDtypeStruct((512, 128), jnp.float32)
    return pl.pallas_call(body, out_shape=(sh, sh, sh))(degp, x_flat)


def _tc_prep2(a1p, d, d2):
    def body(a1p_ref, d_ref, d2_ref, u2_ref, da1_ref):
        a1 = a1p_ref[0] + a1p_ref[1]
        u2_ref[...] = d2_ref[...] * a1
        da1_ref[...] = d_ref[...] * a1

    sh = jax.ShapeDtypeStruct((512, 128), jnp.float32)
    return pl.pallas_call(body, out_shape=(sh, sh))(a1p, d, d2)


def _tc_layer1(x, da1, a2p, d, W1, b1):
    BN = 4096
    G = N // BN

    def body(x_ref, da1_ref, a2p_ref, d_ref, W1_ref, b1_ref, h1_ref, u_ref):
        w0 = W1_ref[0]
        w1 = W1_ref[1]
        w2 = W1_ref[2]
        d = d_ref[...]
        xb = x_ref[...]
        tx1 = -da1_ref[...]
        tx2 = 2.0 * (d * (a2p_ref[0] + a2p_ref[1])) - xb
        out = xb * w0 + tx1 * w1 + tx2 * w2 + b1_ref[...]
        h1 = jnp.maximum(out, 0.0)
        h1_ref[...] = h1
        dh = d * h1
        for q in range(4):
            u_ref[q] = dh[:, 16 * q:16 * (q + 1)]

    return pl.pallas_call(
        body,
        grid=(G,),
        in_specs=[
            pl.BlockSpec((BN, 1), lambda i: (i, 0)),
            pl.BlockSpec((BN, 1), lambda i: (i, 0)),
            pl.BlockSpec((2, BN, 1), lambda i: (0, i, 0)),
            pl.BlockSpec((BN, 1), lambda i: (i, 0)),
            pl.BlockSpec((3, 1, 64), lambda i: (0, 0, 0)),
            pl.BlockSpec((1, 64), lambda i: (0, 0)),
        ],
        out_specs=[
            pl.BlockSpec((BN, 64), lambda i: (i, 0)),
            pl.BlockSpec((4, BN, 16), lambda i: (0, i, 0)),
        ],
        out_shape=[
            jax.ShapeDtypeStruct((N, 64), jnp.float32),
            jax.ShapeDtypeStruct((4, N, 16), jnp.float32),
        ],
    )(x, da1, a2p, d, W1, b1)


def _asm16(cp_ref):
    return jnp.concatenate([cp_ref[0, 0], cp_ref[0, 1],
                            cp_ref[1, 0], cp_ref[1, 1]], axis=1)


def _tc_prep3(c1p, d, d2):
    BN = 4096
    G = N // BN

    def body(c1p_ref, d_ref, d2_ref, u_ref, dc1_ref):
        c1 = _asm16(c1p_ref)
        dc1_ref[...] = d_ref[...] * c1
        u2 = d2_ref[...] * c1
        for q in range(4):
            u_ref[q] = u2[:, 16 * q:16 * (q + 1)]

    return pl.pallas_call(
        body,
        grid=(G,),
        in_specs=[
            pl.BlockSpec((2, 2, BN, 16), lambda i: (0, 0, i, 0)),
            pl.BlockSpec((BN, 1), lambda i: (i, 0)),
            pl.BlockSpec((BN, 1), lambda i: (i, 0)),
        ],
        out_specs=[
            pl.BlockSpec((4, BN, 16), lambda i: (0, i, 0)),
            pl.BlockSpec((BN, 64), lambda i: (i, 0)),
        ],
        out_shape=[
            jax.ShapeDtypeStruct((4, N, 16), jnp.float32),
            jax.ShapeDtypeStruct((N, 64), jnp.float32),
        ],
    )(c1p, d, d2)


def _tc_final(h1, dc1, c2p, d, W2, b2):
    BN = 4096
    G = N // BN

    def body(h1_ref, dc1_ref, c2p_ref, d_ref, W2_ref, b2_ref, m_ref):
        h1b = h1_ref[...]
        tx1 = -dc1_ref[...]
        tx2 = 2.0 * (d_ref[...] * _asm16(c2p_ref)) - h1b
        w0 = W2_ref[0]
        w1 = W2_ref[1]
        w2 = W2_ref[2]
        out = jnp.dot(h1b, w0, preferred_element_type=jnp.float32)
        out = out + jnp.dot(tx1, w1, preferred_element_type=jnp.float32)
        out = out + jnp.dot(tx2, w2, preferred_element_type=jnp.float32)
        out = out + b2_ref[...]
        h2 = jnp.maximum(out, 0.0)
        m_ref[...] = jnp.max(h2, axis=1, keepdims=True)

    return pl.pallas_call(
        body,
        grid=(G,),
        in_specs=[
            pl.BlockSpec((BN, 64), lambda i: (i, 0)),
            pl.BlockSpec((BN, 64), lambda i: (i, 0)),
            pl.BlockSpec((2, 2, BN, 16), lambda i: (0, 0, i, 0)),
            pl.BlockSpec((BN, 1), lambda i: (i, 0)),
            pl.BlockSpec((3, 64, 128), lambda i: (0, 0, 0)),
            pl.BlockSpec((1, 128), lambda i: (0, 0)),
        ],
        out_specs=pl.BlockSpec((BN, 1), lambda i: (i, 0)),
        out_shape=jax.ShapeDtypeStruct((N, 1), jnp.float32),
    )(h1, dc1, c2p, d, W2, b2)


def _tc_head(m2, fc1_W, fc1_b, fc2_W, fc2_b):
    def body(m_ref, w1_ref, b1_ref, w2_ref, b2_ref, o_ref):
        h = jnp.dot(m_ref[...], w1_ref[...], preferred_element_type=jnp.float32)
        h = jnp.maximum(h + b1_ref[...], 0.0)
        o_ref[...] = jnp.dot(h, w2_ref[...],
                             preferred_element_type=jnp.float32) + b2_ref[...]

    return pl.pallas_call(
        body, out_shape=jax.ShapeDtypeStruct((512, 1), jnp.float32),
    )(m2, fc1_W, fc1_b, fc2_W, fc2_b)


def kernel(x, edge_index, W1, b1, W2, b2, fc1_W, fc1_b, fc2_W, fc2_b):
    ei = edge_index
    x1 = x.reshape(512, 128)

    degp = _sc_prop_f1(jnp.ones((N,), jnp.float32), ei, grow=1, srow=0)
    d_f, d2_f, u1_f = _tc_prep1(degp.reshape(2, 512, 128), x1)

    a1p = _sc_prop_f1(u1_f.reshape(N), ei, grow=0, srow=1)
    u2_f, da1_f = _tc_prep2(a1p.reshape(2, 512, 128), d_f, d2_f)

    a2p = _sc_prop_f1(u2_f.reshape(N), ei, grow=0, srow=1)

    d_c = d_f.reshape(N, 1)
    h1, U1 = _tc_layer1(x, da1_f.reshape(N, 1), a2p.reshape(2, N, 1),
                        d_c, W1, b1.reshape(1, 64))

    z8 = jnp.zeros((8, 16), jnp.float32)
    c1p = _sc_prop_f64(jnp.concatenate([U1.reshape(4 * N, 16), z8]), ei)
    U2, dc1 = _tc_prep3(c1p, d_c, d2_f.reshape(N, 1))

    c2p = _sc_prop_f64(jnp.concatenate([U2.reshape(4 * N, 16), z8]), ei)

    m = _tc_final(h1, dc1, c2p, d_c, W2, b2.reshape(1, 128))
    return _tc_head(m.reshape(512, 128), fc1_W, fc1_b.reshape(1, 64),
                    fc2_W, fc2_b.reshape(1, 1))
